# baseline (speedup 1.0000x reference)
import functools

import jax
import jax.numpy as jnp
from jax import lax
from jax.experimental import pallas as pl
from jax.experimental.pallas import tpu as pltpu
from jax.experimental.pallas import tpu_sc as plsc

N_NODES = 10000
N_EDGES = 320000
D = 128

NC = 2
NS = 16
NW = NC * NS
E_PER_W = N_EDGES // NW
CHUNK = 100
EPW_PAD = 10000
NKW = EPW_PAD // CHUNK
GRP = 4
NG = NKW // GRP
N_PAD = 10240
ROWS_PER_TILE = N_PAD // NS


def _sc_segment_sum(x, src_w, dst_w, zeros):
    mesh = plsc.VectorSubcoreMesh(
        core_axis_name="c", subcore_axis_name="s", num_cores=NC, num_subcores=NS
    )

    @functools.partial(
        pl.kernel,
        out_type=jax.ShapeDtypeStruct((NC, N_PAD, D), jnp.float32),
        mesh=mesh,
        scratch_types=[
            pltpu.VMEM((2, GRP, CHUNK), jnp.int32),
            pltpu.VMEM((2, GRP, CHUNK), jnp.int32),
            pltpu.VMEM((CHUNK, D), jnp.float32),
            pltpu.VMEM((CHUNK, D), jnp.float32),
            pltpu.VMEM_SHARED((N_PAD, D), jnp.float32),
            pltpu.SemaphoreType.DMA,
            pltpu.SemaphoreType.DMA,
            pltpu.SemaphoreType.DMA,
            pltpu.SemaphoreType.DMA,
            pltpu.SemaphoreType.DMA,
        ],
    )
    def agg_kernel(x_hbm, src_hbm, dst_hbm, zeros_hbm, out_hbm,
                   src_v, dst_v, rows0_v, rows1_v, acc_sh,
                   sem0, sem1, sems0, sems1, semi):
        c = lax.axis_index("c")
        s = lax.axis_index("s")
        wid = s * NC + c

        row0 = s * ROWS_PER_TILE
        pltpu.sync_copy(
            zeros_hbm.at[pl.ds(0, ROWS_PER_TILE)],
            acc_sh.at[pl.ds(row0, ROWS_PER_TILE)],
        )

        pltpu.sync_copy(src_hbm.at[wid * NG], src_v.at[0])
        pltpu.sync_copy(dst_hbm.at[wid * NG], dst_v.at[0])
        plsc.subcore_barrier()
        pltpu.async_copy(x_hbm.at[src_v.at[0, 0]], rows0_v, sem0)

        rows = (rows0_v, rows1_v)
        gsems = (sem0, sem1)
        ssems = (sems0, sems1)

        @pl.loop(0, NG)
        def group(g):
            p = lax.rem(g, 2)

            @pl.when(g > 0)
            def _():
                pltpu.make_async_copy(
                    rows[(GRP - 1) % 2],
                    acc_sh.at[dst_v.at[1 - p, GRP - 1]],
                    ssems[(GRP - 1) % 2]).wait()

            @pl.when(g < NG - 1)
            def _():
                pltpu.async_copy(src_hbm.at[wid * NG + g + 1], src_v.at[1 - p],
                                 semi)
                pltpu.async_copy(dst_hbm.at[wid * NG + g + 1], dst_v.at[1 - p],
                                 semi)

            for k in range(GRP):
                cur, nxt = rows[k % 2], rows[(k + 1) % 2]
                gsem_cur, gsem_nxt = gsems[k % 2], gsems[(k + 1) % 2]
                if 0 < k < GRP - 1:
                    pltpu.make_async_copy(
                        nxt, acc_sh.at[dst_v.at[p, k - 1]],
                        ssems[(k - 1) % 2]).wait()
                if k < GRP - 1:
                    pltpu.async_copy(x_hbm.at[src_v.at[p, k + 1]], nxt, gsem_nxt)
                else:
                    @pl.when(g < NG - 1)
                    def _():
                        pltpu.make_async_copy(
                            nxt, acc_sh.at[dst_v.at[p, k - 1]],
                            ssems[(k - 1) % 2]).wait()
                        pltpu.make_async_copy(
                            src_hbm.at[wid * NG + g + 1],
                            src_v.at[1 - p], semi).wait()
                        pltpu.make_async_copy(
                            dst_hbm.at[wid * NG + g + 1],
                            dst_v.at[1 - p], semi).wait()
                        pltpu.async_copy(
                            x_hbm.at[src_v.at[1 - p, 0]], nxt, gsem_nxt)
                pltpu.make_async_copy(
                    x_hbm.at[src_v.at[p, k]], cur, gsem_cur).wait()
                pltpu.async_copy(cur, acc_sh.at[dst_v.at[p, k]],
                                 ssems[k % 2], add=True)

        p_last = (NG - 1) % 2
        pltpu.make_async_copy(
            rows[(GRP - 2) % 2], acc_sh.at[dst_v.at[p_last, GRP - 2]],
            ssems[(GRP - 2) % 2]).wait()
        pltpu.make_async_copy(
            rows[(GRP - 1) % 2], acc_sh.at[dst_v.at[p_last, GRP - 1]],
            ssems[(GRP - 1) % 2]).wait()

        plsc.subcore_barrier()
        pltpu.sync_copy(
            acc_sh.at[pl.ds(row0, ROWS_PER_TILE)],
            out_hbm.at[c].at[pl.ds(row0, ROWS_PER_TILE)],
        )

    return agg_kernel(x, src_w, dst_w, zeros)


def _self_kernel(x_ref, ws_ref, b_ref, o_ref):
    o_ref[...] = (
        jnp.dot(x_ref[...], ws_ref[...], preferred_element_type=jnp.float32)
        + b_ref[...])


def _self_part(x, W_self, b):
    blk = 2000
    return pl.pallas_call(
        _self_kernel,
        out_shape=jax.ShapeDtypeStruct((N_NODES, D), jnp.float32),
        grid=(N_NODES // blk,),
        in_specs=[
            pl.BlockSpec((blk, D), lambda i: (i, 0)),
            pl.BlockSpec((D, D), lambda i: (0, 0)),
            pl.BlockSpec((1, D), lambda i: (0, 0)),
        ],
        out_specs=pl.BlockSpec((blk, D), lambda i: (i, 0)),
    )(x, W_self, b)


def _nbr_kernel(h1_ref, a0_ref, a1_ref, wn_ref, o_ref):
    agg = a0_ref[0] + a1_ref[0]
    h = h1_ref[...] + jnp.dot(agg, wn_ref[...],
                              preferred_element_type=jnp.float32)
    o_ref[...] = jnp.maximum(h, 0.0)


def _nbr_part(h1, agg, W_nbr):
    blk = 2000
    return pl.pallas_call(
        _nbr_kernel,
        out_shape=jax.ShapeDtypeStruct((N_NODES, D), jnp.float32),
        grid=(N_NODES // blk,),
        in_specs=[
            pl.BlockSpec((blk, D), lambda i: (i, 0)),
            pl.BlockSpec((1, blk, D), lambda i: (0, i, 0)),
            pl.BlockSpec((1, blk, D), lambda i: (1, i, 0)),
            pl.BlockSpec((D, D), lambda i: (0, 0)),
        ],
        out_specs=pl.BlockSpec((blk, D), lambda i: (i, 0)),
    )(h1, agg, agg, W_nbr)


@jax.jit
def kernel(x, edge_index, W_self, W_nbr, b):
    ei = edge_index.astype(jnp.int32)
    pad = EPW_PAD - E_PER_W
    src_w = jnp.pad(ei[0].reshape(NW, E_PER_W), ((0, 0), (0, pad)),
                    constant_values=0).reshape(NW * NG, GRP, CHUNK)
    dummy = jnp.broadcast_to(N_NODES + jnp.arange(pad, dtype=jnp.int32),
                             (NW, pad))
    dst_w = jnp.concatenate(
        [ei[1].reshape(NW, E_PER_W), dummy], axis=1).reshape(NW * NG, GRP, CHUNK)
    zeros = jnp.zeros((ROWS_PER_TILE, D), dtype=jnp.float32)
    agg = _sc_segment_sum(x, src_w, dst_w, zeros)
    h1 = _self_part(x, W_self, b.reshape(1, D))
    return _nbr_part(h1, agg, W_nbr)


# speedup vs baseline: 1.0176x; 1.0176x over previous
import functools

import jax
import jax.numpy as jnp
from jax import lax
from jax.experimental import pallas as pl
from jax.experimental.pallas import tpu as pltpu
from jax.experimental.pallas import tpu_sc as plsc

N_NODES = 10000
N_EDGES = 320000
D = 128

NC = 2
NS = 16
NW = NC * NS
E_PER_W = N_EDGES // NW
CHUNK = 100
EPW_PAD = 10000
NKW = EPW_PAD // CHUNK
GRP = 4
NG = NKW // GRP
N_PAD = 10240
ROWS_PER_TILE = N_PAD // NS


def _sc_segment_sum(x, src_w, dst_w, zeros):
    mesh = plsc.VectorSubcoreMesh(
        core_axis_name="c", subcore_axis_name="s", num_cores=NC, num_subcores=NS
    )

    @functools.partial(
        pl.kernel,
        out_type=jax.ShapeDtypeStruct((NC, N_PAD, D), jnp.float32),
        mesh=mesh,
        scratch_types=[
            pltpu.VMEM((2, GRP, CHUNK), jnp.int32),
            pltpu.VMEM((2, GRP, CHUNK), jnp.int32),
            pltpu.VMEM((CHUNK, D), jnp.float32),
            pltpu.VMEM((CHUNK, D), jnp.float32),
            pltpu.VMEM_SHARED((N_PAD, D), jnp.float32),
            pltpu.SemaphoreType.DMA,
            pltpu.SemaphoreType.DMA,
            pltpu.SemaphoreType.DMA,
            pltpu.SemaphoreType.DMA,
            pltpu.SemaphoreType.DMA,
        ],
    )
    def agg_kernel(x_hbm, src_hbm, dst_hbm, zeros_hbm, out_hbm,
                   src_v, dst_v, rows0_v, rows1_v, acc_sh,
                   sem0, sem1, sems0, sems1, semi):
        c = lax.axis_index("c")
        s = lax.axis_index("s")
        wid = s * NC + c

        row0 = s * ROWS_PER_TILE
        pltpu.sync_copy(
            zeros_hbm.at[pl.ds(0, ROWS_PER_TILE)],
            acc_sh.at[pl.ds(row0, ROWS_PER_TILE)],
        )

        pltpu.sync_copy(src_hbm.at[wid * NG], src_v.at[0])
        pltpu.sync_copy(dst_hbm.at[wid * NG], dst_v.at[0])
        plsc.subcore_barrier()
        pltpu.async_copy(x_hbm.at[src_v.at[0, 0]], rows0_v, sem0)

        rows = (rows0_v, rows1_v)
        gsems = (sem0, sem1)
        ssems = (sems0, sems1)

        @pl.loop(0, NG)
        def group(g):
            p = lax.rem(g, 2)

            @pl.when(g > 0)
            def _():
                pltpu.make_async_copy(
                    rows[(GRP - 1) % 2],
                    acc_sh.at[dst_v.at[1 - p, GRP - 1]],
                    ssems[(GRP - 1) % 2]).wait()

            @pl.when(g < NG - 1)
            def _():
                pltpu.async_copy(src_hbm.at[wid * NG + g + 1], src_v.at[1 - p],
                                 semi)
                pltpu.async_copy(dst_hbm.at[wid * NG + g + 1], dst_v.at[1 - p],
                                 semi)

            for k in range(GRP):
                cur, nxt = rows[k % 2], rows[(k + 1) % 2]
                gsem_cur, gsem_nxt = gsems[k % 2], gsems[(k + 1) % 2]
                if 0 < k < GRP - 1:
                    pltpu.make_async_copy(
                        nxt, acc_sh.at[dst_v.at[p, k - 1]],
                        ssems[(k - 1) % 2]).wait()
                if k < GRP - 1:
                    pltpu.async_copy(x_hbm.at[src_v.at[p, k + 1]], nxt, gsem_nxt)
                else:
                    @pl.when(g < NG - 1)
                    def _():
                        pltpu.make_async_copy(
                            nxt, acc_sh.at[dst_v.at[p, k - 1]],
                            ssems[(k - 1) % 2]).wait()
                        pltpu.make_async_copy(
                            src_hbm.at[wid * NG + g + 1],
                            src_v.at[1 - p], semi).wait()
                        pltpu.make_async_copy(
                            dst_hbm.at[wid * NG + g + 1],
                            dst_v.at[1 - p], semi).wait()
                        pltpu.async_copy(
                            x_hbm.at[src_v.at[1 - p, 0]], nxt, gsem_nxt)
                pltpu.make_async_copy(
                    x_hbm.at[src_v.at[p, k]], cur, gsem_cur).wait()
                pltpu.async_copy(cur, acc_sh.at[dst_v.at[p, k]],
                                 ssems[k % 2], add=True)

        p_last = (NG - 1) % 2
        pltpu.make_async_copy(
            rows[(GRP - 2) % 2], acc_sh.at[dst_v.at[p_last, GRP - 2]],
            ssems[(GRP - 2) % 2]).wait()
        pltpu.make_async_copy(
            rows[(GRP - 1) % 2], acc_sh.at[dst_v.at[p_last, GRP - 1]],
            ssems[(GRP - 1) % 2]).wait()

        plsc.subcore_barrier()
        pltpu.sync_copy(
            acc_sh.at[pl.ds(row0, ROWS_PER_TILE)],
            out_hbm.at[c].at[pl.ds(row0, ROWS_PER_TILE)],
        )

    return agg_kernel(x, src_w, dst_w, zeros)


def _dense_kernel(x_ref, a0_ref, a1_ref, ws_ref, wn_ref, b_ref, o_ref):
    agg = a0_ref[0] + a1_ref[0]
    h = jnp.dot(x_ref[...], ws_ref[...], preferred_element_type=jnp.float32)
    h = h + jnp.dot(agg, wn_ref[...], preferred_element_type=jnp.float32)
    o_ref[...] = jnp.maximum(h + b_ref[...], 0.0)


def _dense(x, agg, W_self, W_nbr, b):
    blk = 2000
    return pl.pallas_call(
        _dense_kernel,
        out_shape=jax.ShapeDtypeStruct((N_NODES, D), jnp.float32),
        grid=(N_NODES // blk,),
        in_specs=[
            pl.BlockSpec((blk, D), lambda i: (i, 0)),
            pl.BlockSpec((1, blk, D), lambda i: (0, i, 0)),
            pl.BlockSpec((1, blk, D), lambda i: (1, i, 0)),
            pl.BlockSpec((D, D), lambda i: (0, 0)),
            pl.BlockSpec((D, D), lambda i: (0, 0)),
            pl.BlockSpec((1, D), lambda i: (0, 0)),
        ],
        out_specs=pl.BlockSpec((blk, D), lambda i: (i, 0)),
    )(x, agg, agg, W_self, W_nbr, b)


@jax.jit
def kernel(x, edge_index, W_self, W_nbr, b):
    ei = edge_index.astype(jnp.int32)
    pad = EPW_PAD - E_PER_W
    src_w = jnp.pad(ei[0].reshape(NW, E_PER_W), ((0, 0), (0, pad)),
                    constant_values=0).reshape(NW * NG, GRP, CHUNK)
    dummy = jnp.broadcast_to(N_NODES + jnp.arange(pad, dtype=jnp.int32),
                             (NW, pad))
    dst_w = jnp.concatenate(
        [ei[1].reshape(NW, E_PER_W), dummy], axis=1).reshape(NW * NG, GRP, CHUNK)
    zeros = jnp.zeros((ROWS_PER_TILE, D), dtype=jnp.float32)
    agg = _sc_segment_sum(x, src_w, dst_w, zeros)
    return _dense(x, agg, W_self, W_nbr, b.reshape(1, D))
